# baseline (speedup 1.0000x reference)
import functools

import jax
import jax.numpy as jnp
from jax import lax
from jax.experimental import pallas as pl
from jax.experimental.pallas import tpu as pltpu
from jax.experimental.pallas import tpu_sc as plsc

_L = 16
_F = 16384
_C = 8192
_D = 4
_UNROLL = 16


def _sc_fma(x1, n1, ts, ta2, tb2):
    info = plsc.get_sparse_core_info()
    nc, ns = info.num_cores, info.num_subcores
    nw = nc * ns
    (total,) = x1.shape
    rows = total // _F
    rpw = rows // nw
    cpw = rpw * (_F // _C)
    cpr = _F // _C

    mesh = plsc.VectorSubcoreMesh(core_axis_name="c", subcore_axis_name="s")

    @functools.partial(
        pl.kernel,
        mesh=mesh,
        out_type=jax.ShapeDtypeStruct((total,), jnp.float32),
        scratch_types=(
            [pltpu.VMEM((rpw,), jnp.int32),
             pltpu.VMEM((rpw, 128), jnp.float32),
             pltpu.VMEM((rpw, 128), jnp.float32)]
            + [pltpu.VMEM((_C,), jnp.float32)] * (3 * _D)
            + [pltpu.SemaphoreType.DMA,
               pltpu.SemaphoreType.DMA,
               pltpu.SemaphoreType.DMA((_D,)),
               pltpu.SemaphoreType.DMA((_D,)),
               pltpu.SemaphoreType.DMA((_D,))]
        ),
    )
    def k(x_hbm, n_hbm, ts_hbm, ta_hbm, tb_hbm, o_hbm, idx_v, av, bv,
          *rest):
        bufs = rest[:3 * _D]
        xb = bufs[0:_D]
        nb = bufs[_D:2 * _D]
        ob = bufs[2 * _D:3 * _D]
        sg_a, sg_b, sx, sn, so = rest[3 * _D:]

        wid = lax.axis_index("s") * nc + lax.axis_index("c")
        r0 = wid * rpw
        base0 = r0 * _F

        pltpu.sync_copy(ts_hbm.at[pl.ds(r0, rpw)], idx_v)
        pltpu.async_copy(ta_hbm.at[idx_v], av, sg_a).wait()
        pltpu.async_copy(tb_hbm.at[idx_v], bv, sg_b).wait()

        def in_copies(c, s):
            off = base0 + c * _C
            cx = pltpu.make_async_copy(
                x_hbm.at[pl.ds(off, _C)], xb[s], sx.at[s])
            cn = pltpu.make_async_copy(
                n_hbm.at[pl.ds(off, _C)], nb[s], sn.at[s])
            return cx, cn

        def out_copy(c, s):
            off = base0 + c * _C
            return pltpu.make_async_copy(
                ob[s], o_hbm.at[pl.ds(off, _C)], so.at[s])

        for p in range(_D):
            cx, cn = in_copies(p, p)
            cx.start()
            cn.start()

        def do_chunk(c, s):
            cx, cn = in_copies(c, s)
            cx.wait()
            cn.wait()

            @pl.when(c >= _D)
            def _():
                out_copy(c - _D, s).wait()

            r = c // cpr
            a16 = av[r, pl.ds(0, _L)]
            b16 = bv[r, pl.ds(0, _L)]
            xs, ns_, os_ = xb[s], nb[s], ob[s]

            def inner(j, carry2):
                base = j * (_UNROLL * _L)
                for u in range(_UNROLL):
                    o = base + u * _L
                    os_[pl.ds(o, _L)] = (
                        a16 * xs[pl.ds(o, _L)] + b16 * ns_[pl.ds(o, _L)])
                return carry2

            lax.fori_loop(0, _C // (_UNROLL * _L), inner, 0)
            out_copy(c, s).start()

            @pl.when(c + _D < cpw)
            def _():
                c2x, c2n = in_copies(c + _D, s)
                c2x.start()
                c2n.start()

        def group_body(i, carry):
            for s in range(_D):
                do_chunk(_D * i + s, s)
            return carry

        lax.fori_loop(0, cpw // _D, group_body, 0)
        for p in range(cpw - _D, cpw):
            out_copy(p, p % _D).wait()

    return k(x1, n1, ts, ta2, tb2)


def kernel(original_samples, noise, timesteps, sqrt_alphas_cumprod,
           sqrt_one_minus_alphas_cumprod):
    shape = original_samples.shape
    ts = timesteps.astype(jnp.int32)
    ta2 = jnp.broadcast_to(sqrt_alphas_cumprod[:, None], (1000, 128))
    tb2 = jnp.broadcast_to(sqrt_one_minus_alphas_cumprod[:, None], (1000, 128))
    x1 = original_samples.reshape(-1)
    n1 = noise.reshape(-1)
    out = _sc_fma(x1, n1, ts, ta2, tb2)
    return out.reshape(shape)


# speedup vs baseline: 2.0445x; 2.0445x over previous
import functools

import jax
import jax.numpy as jnp
from jax import lax
from jax.experimental import pallas as pl
from jax.experimental.pallas import tpu as pltpu
from jax.experimental.pallas import tpu_sc as plsc

_L = 16


def _sc_gather(table_a, table_b, ts):
    info = plsc.get_sparse_core_info()
    nc, ns = info.num_cores, info.num_subcores
    nw = nc * ns
    (B,) = ts.shape
    bpw = B // nw

    mesh = plsc.VectorSubcoreMesh(core_axis_name="c", subcore_axis_name="s")

    @functools.partial(
        pl.kernel,
        mesh=mesh,
        out_type=[
            jax.ShapeDtypeStruct((B,), jnp.float32),
            jax.ShapeDtypeStruct((B,), jnp.float32),
        ],
        scratch_types=[
            pltpu.VMEM((bpw,), jnp.int32),
            pltpu.VMEM((bpw,), jnp.float32),
            pltpu.VMEM((bpw,), jnp.float32),
            pltpu.SemaphoreType.DMA,
            pltpu.SemaphoreType.DMA,
        ],
    )
    def gather_k(ta_hbm, tb_hbm, ts_hbm, oa_hbm, ob_hbm,
                 idx_v, oa_v, ob_v, sem_a, sem_b):
        wid = lax.axis_index("s") * nc + lax.axis_index("c")
        base = wid * bpw
        pltpu.sync_copy(ts_hbm.at[pl.ds(base, bpw)], idx_v)
        ca = pltpu.async_copy(ta_hbm.at[idx_v], oa_v, sem_a)
        cb = pltpu.async_copy(tb_hbm.at[idx_v], ob_v, sem_b)
        ca.wait()
        cb.wait()
        pltpu.sync_copy(oa_v, oa_hbm.at[pl.ds(base, bpw)])
        pltpu.sync_copy(ob_v, ob_hbm.at[pl.ds(base, bpw)])

    return gather_k(table_a, table_b, ts)


def _make_fma_body(block_b):
    def _fma_body(x_ref, n_ref, a_ref, b_ref, o_ref):
        i = pl.program_id(0)
        a = a_ref[pl.ds(i * block_b, block_b), :]
        b = b_ref[pl.ds(i * block_b, block_b), :]
        o_ref[...] = a * x_ref[...] + b * n_ref[...]
    return _fma_body


def _tc_fma(x, n, a, b, block_b=64):
    M, W = x.shape
    return pl.pallas_call(
        _make_fma_body(block_b),
        grid=(M // block_b,),
        in_specs=[
            pl.BlockSpec((block_b, W), lambda i: (i, 0)),
            pl.BlockSpec((block_b, W), lambda i: (i, 0)),
            pl.BlockSpec((M, 1), lambda i: (0, 0)),
            pl.BlockSpec((M, 1), lambda i: (0, 0)),
        ],
        out_specs=pl.BlockSpec((block_b, W), lambda i: (i, 0)),
        out_shape=jax.ShapeDtypeStruct((M, W), jnp.float32),
    )(x, n, a, b)


def kernel(original_samples, noise, timesteps, sqrt_alphas_cumprod,
           sqrt_one_minus_alphas_cumprod):
    shape = original_samples.shape
    B = shape[0]
    ts = timesteps.astype(jnp.int32)
    T = sqrt_alphas_cumprod.shape[0]
    pad = (-T) % _L
    ta = jnp.pad(sqrt_alphas_cumprod, (0, pad))
    tb = jnp.pad(sqrt_one_minus_alphas_cumprod, (0, pad))
    a, b = _sc_gather(ta, tb, ts)
    x2 = original_samples.reshape(B, -1)
    n2 = noise.reshape(B, -1)
    out = _tc_fma(x2, n2, a.reshape(B, 1), b.reshape(B, 1))
    return out.reshape(shape)
